# baseline (speedup 1.0000x reference)
import functools

import jax
import jax.numpy as jnp
from jax import lax
from jax.experimental import pallas as pl
from jax.experimental.pallas import tpu as pltpu
from jax.experimental.pallas import tpu_sc as plsc

DIM = 1024
NUM_EXPERTS = 4
TOP_K = 2
ADJ_HIDDEN = 1368
N_TOKENS = 2048

TB2 = 256
NBS = (N_TOKENS * TOP_K) // TB2 + NUM_EXPERTS
A_PAD = NBS * TB2

NC = 2
NS = 16
NW = NC * NS
TPW = N_TOKENS // NW
HALF = TPW // 2


def _router_body(x_ref, wg_ref, pos_ref, wb_ref, bexp_ref):
    x = x_ref[...]
    logits = jax.lax.dot_general(
        x, wg_ref[...], (((1,), (1,)), ((), ())),
        preferred_element_type=jnp.float32,
        precision=jax.lax.Precision.DEFAULT)
    m = jnp.max(logits, axis=-1, keepdims=True)
    ex = jnp.exp(logits - m)
    probs = ex / jnp.sum(ex, axis=-1, keepdims=True)

    cols = [probs[:, c] for c in range(NUM_EXPERTS)]
    ranks = []
    for c in range(NUM_EXPERTS):
        rank = jnp.zeros_like(cols[c])
        for j in range(NUM_EXPERTS):
            if j == c:
                continue
            beats = (cols[j] > cols[c]) | ((cols[j] == cols[c]) & (j < c))
            rank = rank + beats.astype(jnp.float32)
        ranks.append(rank)
    sel = [(r < TOP_K).astype(jnp.float32) for r in ranks]
    sum_sel = sum(s * p for s, p in zip(sel, cols))
    tw = [s * p / (sum_sel + 1e-8) for s, p in zip(sel, cols)]
    mask0 = [(r == 0).astype(jnp.float32) for r in ranks]
    mask1 = [(r == 1).astype(jnp.float32) for r in ranks]

    s_all = jnp.stack(sel, axis=-1)
    CH = 256
    row = jax.lax.broadcasted_iota(jnp.int32, (CH, CH), 0)
    col = jax.lax.broadcasted_iota(jnp.int32, (CH, CH), 1)
    lt = (col < row).astype(jnp.float32)
    off = jnp.zeros((1, NUM_EXPERTS), jnp.float32)
    chunks = []
    for i in range(N_TOKENS // CH):
        sc = s_all[i * CH:(i + 1) * CH, :]
        chunks.append(jax.lax.dot_general(
            lt, sc, (((1,), (0,)), ((), ())),
            preferred_element_type=jnp.float32,
            precision=jax.lax.Precision.DEFAULT) + off)
        off = off + jnp.sum(sc, axis=0, keepdims=True)
    rwe = jnp.concatenate(chunks, axis=0)
    cnt = off

    pcnt = jnp.ceil(cnt / TB2) * TB2
    eidx_r = jax.lax.broadcasted_iota(jnp.int32, (NUM_EXPERTS, NUM_EXPERTS), 0)
    eidx_c = jax.lax.broadcasted_iota(jnp.int32, (NUM_EXPERTS, NUM_EXPERTS), 1)
    su = (eidx_r < eidx_c).astype(jnp.float32)
    base = jax.lax.dot_general(
        pcnt, su, (((1,), (0,)), ((), ())),
        preferred_element_type=jnp.float32,
        precision=jax.lax.Precision.DEFAULT)

    pos0 = sum(mask0[c] * (base[0, c] + rwe[:, c]) for c in range(NUM_EXPERTS))
    pos1 = sum(mask1[c] * (base[0, c] + rwe[:, c]) for c in range(NUM_EXPERTS))
    pos_ref[...] = jnp.stack([pos0, pos1], axis=0).astype(jnp.int32)

    m0tw = jnp.stack([mask0[c] * tw[c] for c in range(NUM_EXPERTS)], axis=-1)
    m1tw = jnp.stack([mask1[c] * tw[c] for c in range(NUM_EXPERTS)], axis=-1)
    ones = jnp.ones((NUM_EXPERTS, 128), jnp.float32)
    wb_ref[0] = jax.lax.dot_general(
        m0tw, ones, (((1,), (0,)), ((), ())),
        preferred_element_type=jnp.float32,
        precision=jax.lax.Precision.HIGHEST)
    wb_ref[1] = jax.lax.dot_general(
        m1tw, ones, (((1,), (0,)), ((), ())),
        preferred_element_type=jnp.float32,
        precision=jax.lax.Precision.HIGHEST)

    blk_start = jax.lax.broadcasted_iota(jnp.int32, (NBS, 1), 0).astype(jnp.float32) * TB2
    seg_end = base + pcnt
    exp_blk = jnp.sum((blk_start >= seg_end).astype(jnp.int32), axis=1)
    exp_blk = jnp.minimum(exp_blk, NUM_EXPERTS - 1)
    total_used = jnp.sum(pcnt)
    valid = (blk_start[:, 0] < total_used).astype(jnp.int32)
    bexp_ref[...] = jnp.stack([exp_blk, valid], axis=0)


def _router(x, Wg):
    return pl.pallas_call(
        _router_body,
        out_shape=[
            jax.ShapeDtypeStruct((2, N_TOKENS), jnp.int32),
            jax.ShapeDtypeStruct((2, N_TOKENS, 128), jnp.float32),
            jax.ShapeDtypeStruct((2, NBS), jnp.int32),
        ],
    )(x, Wg)


def _dispatch_body(x_hbm, pos_hbm, wb_hbm, xg_hbm, ws_hbm,
                   i0, i1, xbuf, w0b, w1b, sem):
    wid = lax.axis_index("s") * NC + lax.axis_index("c")
    base = wid * TPW
    pltpu.sync_copy(pos_hbm.at[0, pl.ds(base, TPW)], i0)
    pltpu.sync_copy(pos_hbm.at[1, pl.ds(base, TPW)], i1)
    pltpu.sync_copy(x_hbm.at[pl.ds(base, TPW)], xbuf)
    pltpu.sync_copy(wb_hbm.at[0, pl.ds(base, TPW), :], w0b)
    pltpu.sync_copy(wb_hbm.at[1, pl.ds(base, TPW), :], w1b)
    pltpu.async_copy(xbuf, xg_hbm.at[i0], sem).wait()
    pltpu.async_copy(xbuf, xg_hbm.at[i1], sem).wait()
    pltpu.async_copy(w0b, ws_hbm.at[i0], sem).wait()
    pltpu.async_copy(w1b, ws_hbm.at[i1], sem).wait()


def _dispatch(x, pos, wb):
    mesh = plsc.VectorSubcoreMesh(core_axis_name="c", subcore_axis_name="s")
    f = functools.partial(
        pl.kernel,
        mesh=mesh,
        out_type=[
            jax.ShapeDtypeStruct((A_PAD, DIM), jnp.float32),
            jax.ShapeDtypeStruct((A_PAD, 128), jnp.float32),
        ],
        scratch_types=[
            pltpu.VMEM((TPW,), jnp.int32),
            pltpu.VMEM((TPW,), jnp.int32),
            pltpu.VMEM((TPW, DIM), jnp.float32),
            pltpu.VMEM((TPW, 128), jnp.float32),
            pltpu.VMEM((TPW, 128), jnp.float32),
            pltpu.SemaphoreType.DMA,
        ],
    )(_dispatch_body)
    return f(x, pos, wb)


def _mm_body(bexp_ref, xg_ref, ws_ref, wgate_ref, w1_ref, w2_ref, out_ref):
    b = pl.program_id(0)

    @pl.when(bexp_ref[1, b] == 1)
    def _():
        xb = xg_ref[...]
        g = jax.lax.dot_general(
            xb, wgate_ref[0], (((1,), (1,)), ((), ())),
            preferred_element_type=jnp.float32,
            precision=jax.lax.Precision.DEFAULT)
        u = jax.lax.dot_general(
            xb, w1_ref[0], (((1,), (1,)), ((), ())),
            preferred_element_type=jnp.float32,
            precision=jax.lax.Precision.DEFAULT)
        gu = g * jax.nn.sigmoid(g) * u
        eo = jax.lax.dot_general(
            gu, w2_ref[0], (((1,), (1,)), ((), ())),
            preferred_element_type=jnp.float32,
            precision=jax.lax.Precision.DEFAULT)
        out_ref[...] = ws_ref[:, 0:1] * eo


def _grouped_mm(bexp, xg, ws, W_gate, W1, W2):
    grid_spec = pltpu.PrefetchScalarGridSpec(
        num_scalar_prefetch=1,
        grid=(NBS,),
        in_specs=[
            pl.BlockSpec((TB2, DIM), lambda b, bexp: (b, 0)),
            pl.BlockSpec((TB2, 128), lambda b, bexp: (b, 0)),
            pl.BlockSpec((1, ADJ_HIDDEN, DIM), lambda b, bexp: (bexp[0, b], 0, 0)),
            pl.BlockSpec((1, ADJ_HIDDEN, DIM), lambda b, bexp: (bexp[0, b], 0, 0)),
            pl.BlockSpec((1, DIM, ADJ_HIDDEN), lambda b, bexp: (bexp[0, b], 0, 0)),
        ],
        out_specs=pl.BlockSpec((TB2, DIM), lambda b, bexp: (b, 0)),
    )
    return pl.pallas_call(
        _mm_body,
        grid_spec=grid_spec,
        out_shape=jax.ShapeDtypeStruct((A_PAD, DIM), jnp.float32),
        compiler_params=pltpu.CompilerParams(
            dimension_semantics=("arbitrary",),
        ),
    )(bexp, xg, ws, W_gate, W1, W2)


def _combine_body(eo_hbm, pos_hbm, out_hbm, i0, i1, e0, e1, ob, sem):
    wid = lax.axis_index("s") * NC + lax.axis_index("c")
    base = wid * TPW
    for h in range(TPW // HALF):
        hbase = base + h * HALF
        pltpu.sync_copy(pos_hbm.at[0, pl.ds(hbase, HALF)], i0)
        pltpu.sync_copy(pos_hbm.at[1, pl.ds(hbase, HALF)], i1)
        pltpu.async_copy(eo_hbm.at[i0], e0, sem).wait()
        pltpu.async_copy(eo_hbm.at[i1], e1, sem).wait()

        def body(t, carry):
            for c in range(0, DIM, 16):
                ob[t, pl.ds(c, 16)] = e0[t, pl.ds(c, 16)] + e1[t, pl.ds(c, 16)]
            return carry

        lax.fori_loop(0, HALF, body, 0)
        pltpu.sync_copy(ob, out_hbm.at[pl.ds(hbase, HALF)])


def _combine(eo, pos):
    mesh = plsc.VectorSubcoreMesh(core_axis_name="c", subcore_axis_name="s")
    f = functools.partial(
        pl.kernel,
        mesh=mesh,
        out_type=jax.ShapeDtypeStruct((N_TOKENS, DIM), jnp.float32),
        scratch_types=[
            pltpu.VMEM((HALF,), jnp.int32),
            pltpu.VMEM((HALF,), jnp.int32),
            pltpu.VMEM((HALF, DIM), jnp.float32),
            pltpu.VMEM((HALF, DIM), jnp.float32),
            pltpu.VMEM((HALF, DIM), jnp.float32),
            pltpu.SemaphoreType.DMA,
        ],
    )(_combine_body)
    return f(eo, pos)


def kernel(x, Wg, W_gate, W1, W2):
    pos, wb, bexp = _router(x, Wg)
    xg, ws = _dispatch(x, pos, wb)
    eo = _grouped_mm(bexp, xg, ws, W_gate, W1, W2)
    return _combine(eo, pos)
